# Initial kernel scaffold; baseline (speedup 1.0000x reference)
import jax
import jax.numpy as jnp
from jax.experimental import pallas as pl


def kernel(h, e, edge_index, emb_h_W, emb_h_b, emb_e_W, emb_e_b, layer_W, layer_b, bn_gamma, bn_beta):
    raise NotImplementedError("write your pallas kernel here")


# speedup vs baseline: 50.1604x; 50.1604x over previous
import functools

import jax
import jax.numpy as jnp
from jax import lax
from jax.experimental import pallas as pl
from jax.experimental.pallas import tpu as pltpu
from jax.experimental.pallas import tpu_sc as plsc

F32 = jnp.float32


def _z():
    return jnp.int32(0)

NUM_CORES = 2
NUM_SUBCORES = 16
NUM_WORKERS = NUM_CORES * NUM_SUBCORES


def _affine_body(x_ref, w_ref, b_ref, o_ref):
    o_ref[...] = (
        jnp.dot(x_ref[...], w_ref[...], preferred_element_type=F32) + b_ref[...]
    )


def _affine(x, w, b, bm):
    m, k = x.shape
    n = w.shape[1]
    return pl.pallas_call(
        _affine_body,
        grid=(m // bm,),
        in_specs=[
            pl.BlockSpec((bm, k), lambda i: (i, _z())),
            pl.BlockSpec((k, n), lambda i: (_z(), _z())),
            pl.BlockSpec((1, n), lambda i: (_z(), _z())),
        ],
        out_specs=pl.BlockSpec((bm, n), lambda i: (i, _z())),
        out_shape=jax.ShapeDtypeStruct((m, n), F32),
    )(x, w, b.reshape(1, n))


def _mm4_body(h_ref, w_ref, b_ref, db_ref, ea_ref):
    y = jnp.dot(h_ref[...], w_ref[...], preferred_element_type=F32) + b_ref[...]
    hdim = y.shape[1] // 4
    db_ref[...] = y[:, : 2 * hdim]
    ea_ref[...] = y[:, 2 * hdim :]


def _mm4(h, w_cat, b_cat, bm):
    m, k = h.shape
    hd = w_cat.shape[1] // 4
    return pl.pallas_call(
        _mm4_body,
        grid=(m // bm,),
        in_specs=[
            pl.BlockSpec((bm, k), lambda i: (i, _z())),
            pl.BlockSpec((k, 4 * hd), lambda i: (_z(), _z())),
            pl.BlockSpec((1, 4 * hd), lambda i: (_z(), _z())),
        ],
        out_specs=[
            pl.BlockSpec((bm, 2 * hd), lambda i: (i, _z())),
            pl.BlockSpec((bm, 2 * hd), lambda i: (i, _z())),
        ],
        out_shape=[
            jax.ShapeDtypeStruct((m, 2 * hd), F32),
            jax.ShapeDtypeStruct((m, 2 * hd), F32),
        ],
    )(h, w_cat, b_cat.reshape(1, 4 * hd))


def _update_body(ea_ref, acc_ref, hin_ref, sb_ref, ho_ref):
    hd = hin_ref.shape[1]
    ah = ea_ref[:, hd:]
    acc0 = acc_ref[0]
    acc1 = acc_ref[1]
    num = acc0[:, :hd] + acc1[:, :hd]
    den = acc0[:, hd:] + acc1[:, hd:]
    hn = ah + num / (den + 1e-6)
    scale = sb_ref[0:1, :]
    beta = sb_ref[1:2, :]
    ho_ref[...] = jnp.maximum(hn * scale + beta, 0.0) + hin_ref[...]


def _update(ea, accp, h_in, sb, bm):
    m, hd = h_in.shape
    return pl.pallas_call(
        _update_body,
        grid=(m // bm,),
        in_specs=[
            pl.BlockSpec((bm, 2 * hd), lambda i: (i, _z())),
            pl.BlockSpec((2, bm, 2 * hd), lambda i: (_z(), i, _z())),
            pl.BlockSpec((bm, hd), lambda i: (i, _z())),
            pl.BlockSpec((2, hd), lambda i: (_z(), _z())),
        ],
        out_specs=pl.BlockSpec((bm, hd), lambda i: (i, _z())),
        out_shape=jax.ShapeDtypeStruct((m, hd), F32),
    )(ea, accp, h_in, sb)


@functools.lru_cache(maxsize=None)
def _make_edge_kernel(n_edges, n_nodes, hd):
    epw = n_edges // NUM_WORKERS
    chunk = 40
    n_chunks = epw // chunk
    assert epw % chunk == 0 and chunk % 8 == 0
    rbase = (n_nodes // (16 * NUM_SUBCORES)) * 16
    zchunks_base = rbase // 16
    zchunks_last = (n_nodes - rbase * (NUM_SUBCORES - 1)) // 16
    assert n_nodes % 16 == 0

    mesh = plsc.VectorSubcoreMesh(core_axis_name="c", subcore_axis_name="s")

    def body(src_h, dst_h, ce_h, ein_h, db_h, ea_h, prm_h,
             eout_h, accp_h,
             srcv, dstv, dbv, eav, cev, einv, eoutv, contribv, prmv, accs):
        i32 = jnp.int32
        c = lax.axis_index("c").astype(i32)
        s = lax.axis_index("s").astype(i32)
        w = c * i32(NUM_SUBCORES) + s

        pltpu.sync_copy(prm_h, prmv)

        def zrow(r, carry):
            z = jnp.zeros((16,), F32)
            for j in range(2 * hd // 16):
                contribv[r, pl.ds(j * 16, 16)] = z
            return carry

        lax.fori_loop(i32(0), i32(16), zrow, None)
        row0 = s * i32(rbase)
        nz = jnp.where(s == i32(NUM_SUBCORES - 1),
                       i32(zchunks_last), i32(zchunks_base))

        def zcopy(k, carry):
            pltpu.sync_copy(
                contribv.at[pl.ds(0, 16)],
                accs.at[pl.ds(row0 + k * i32(16), 16)],
            )
            return carry

        lax.fori_loop(i32(0), nz, zcopy, None)
        plsc.subcore_barrier()

        base_w = w * i32(epw)

        def do_chunk(t, carry):
            base = pl.multiple_of(base_w + t * i32(chunk), 8)
            pltpu.sync_copy(src_h.at[pl.ds(base, chunk)], srcv)
            pltpu.sync_copy(dst_h.at[pl.ds(base, chunk)], dstv)
            pltpu.sync_copy(ce_h.at[pl.ds(base, chunk)], cev)
            pltpu.sync_copy(ein_h.at[pl.ds(base, chunk)], einv)
            pltpu.sync_copy(db_h.at[srcv], dbv)
            pltpu.sync_copy(ea_h.at[dstv], eav)

            def row(r, rc):
                for j in range(hd // 16):
                    o = j * 16
                    dh = dbv[r, pl.ds(o, 16)]
                    bh = dbv[r, pl.ds(hd + o, 16)]
                    ehg = eav[r, pl.ds(o, 16)]
                    ce = cev[r, pl.ds(o, 16)]
                    tv = ce + dh + ehg
                    sg = 1.0 / (1.0 + jnp.exp(-tv))
                    contribv[r, pl.ds(o, 16)] = sg * bh
                    contribv[r, pl.ds(hd + o, 16)] = sg
                    scl = prmv[pl.ds(o, 16)]
                    bt = prmv[pl.ds(hd + o, 16)]
                    ei = einv[r, pl.ds(o, 16)]
                    eoutv[r, pl.ds(o, 16)] = (
                        jnp.maximum(scl * tv + bt, 0.0) + ei
                    )
                return rc

            lax.fori_loop(i32(0), i32(chunk), row, None)

            pltpu.sync_copy(eoutv, eout_h.at[pl.ds(base, chunk)])
            pltpu.sync_copy(contribv, accs.at[dstv], add=True)
            return carry

        lax.fori_loop(i32(0), i32(n_chunks), do_chunk, None)
        plsc.subcore_barrier()

        def wcopy(k, carry):
            r = row0 + k * i32(16)
            pltpu.sync_copy(
                accs.at[pl.ds(r, 16)],
                accp_h.at[c, pl.ds(r, 16)],
            )
            return carry

        lax.fori_loop(i32(0), nz, wcopy, None)

    return pl.kernel(
        body,
        mesh=mesh,
        out_type=[
            jax.ShapeDtypeStruct((n_edges, hd), F32),
            jax.ShapeDtypeStruct((NUM_CORES, n_nodes, 2 * hd), F32),
        ],
        scratch_types=[
            pltpu.VMEM((chunk,), jnp.int32),
            pltpu.VMEM((chunk,), jnp.int32),
            pltpu.VMEM((chunk, 2 * hd), F32),
            pltpu.VMEM((chunk, 2 * hd), F32),
            pltpu.VMEM((chunk, hd), F32),
            pltpu.VMEM((chunk, hd), F32),
            pltpu.VMEM((chunk, hd), F32),
            pltpu.VMEM((chunk, 2 * hd), F32),
            pltpu.VMEM((2 * hd,), F32),
            pltpu.VMEM_SHARED((n_nodes, 2 * hd), F32),
        ],
    )


def kernel(h, e, edge_index, emb_h_W, emb_h_b, emb_e_W, emb_e_b,
           layer_W, layer_b, bn_gamma, bn_beta):
    n_nodes = h.shape[0]
    n_edges = e.shape[0]
    hd = emb_h_W.shape[1]
    n_layers = layer_W.shape[0]

    src = edge_index[0].astype(jnp.int32)
    dst = edge_index[1].astype(jnp.int32)

    out_dtype = jnp.result_type(h.dtype, emb_h_W.dtype)
    h = h.astype(F32)
    e = e.astype(F32)
    emb_h_W = emb_h_W.astype(F32)
    emb_e_W = emb_e_W.astype(F32)
    emb_h_b = emb_h_b.astype(F32)
    emb_e_b = emb_e_b.astype(F32)
    layer_W = layer_W.astype(F32)
    layer_b = layer_b.astype(F32)
    bn_gamma = bn_gamma.astype(F32)
    bn_beta = bn_beta.astype(F32)

    h = _affine(h, emb_h_W, emb_h_b, 2000)
    e = _affine(e, emb_e_W, emb_e_b, 4000)

    bn_inv = 1.0 / jnp.sqrt(jnp.float32(1.0 + 1e-5))
    edge_fn = _make_edge_kernel(n_edges, n_nodes, hd)

    for l in range(n_layers):
        wl = layer_W[l]
        bl = layer_b[l]
        w_cat = jnp.concatenate([wl[2], wl[1], wl[3], wl[0]], axis=1)
        b_cat = jnp.concatenate([bl[2], bl[1], bl[3], bl[0]])
        db, ea = _mm4(h, w_cat, b_cat, 2000)
        ce = _affine(e, wl[4], bl[4], 4000)

        prm_e = jnp.concatenate([bn_gamma[l, 1] * bn_inv, bn_beta[l, 1]])
        e_new, accp = edge_fn(src, dst, ce, e, db, ea, prm_e)

        sb_h = jnp.stack([bn_gamma[l, 0] * bn_inv, bn_beta[l, 0]])
        h = _update(ea, accp, h, sb_h, 2000)
        e = e_new

    return h.astype(out_dtype), e.astype(out_dtype)


# speedup vs baseline: 83.7280x; 1.6692x over previous
import functools

import jax
import jax.numpy as jnp
from jax import lax
from jax.experimental import pallas as pl
from jax.experimental.pallas import tpu as pltpu
from jax.experimental.pallas import tpu_sc as plsc

F32 = jnp.float32


def _z():
    return jnp.int32(0)

NUM_CORES = 2
NUM_SUBCORES = 16
NUM_WORKERS = NUM_CORES * NUM_SUBCORES


def _affine_body(x_ref, w_ref, b_ref, o_ref):
    o_ref[...] = (
        jnp.dot(x_ref[...], w_ref[...], preferred_element_type=F32) + b_ref[...]
    )


def _affine(x, w, b, bm):
    m, k = x.shape
    n = w.shape[1]
    return pl.pallas_call(
        _affine_body,
        grid=(m // bm,),
        in_specs=[
            pl.BlockSpec((bm, k), lambda i: (i, _z())),
            pl.BlockSpec((k, n), lambda i: (_z(), _z())),
            pl.BlockSpec((1, n), lambda i: (_z(), _z())),
        ],
        out_specs=pl.BlockSpec((bm, n), lambda i: (i, _z())),
        out_shape=jax.ShapeDtypeStruct((m, n), F32),
    )(x, w, b.reshape(1, n))


def _cef_body(t_ref, ein_ref, prm_ref, w_ref, b_ref, ce_ref, eo_ref):
    scale = prm_ref[0:1, :]
    beta = prm_ref[1:2, :]
    eo = jnp.maximum(t_ref[...] * scale + beta, 0.0) + ein_ref[...]
    eo_ref[...] = eo
    ce_ref[...] = jnp.dot(eo, w_ref[...], preferred_element_type=F32) + b_ref[...]


def _ce_fused(t, e_in, prm, w, b, bm):
    m, hd = t.shape
    return pl.pallas_call(
        _cef_body,
        grid=(m // bm,),
        in_specs=[
            pl.BlockSpec((bm, hd), lambda i: (i, _z())),
            pl.BlockSpec((bm, hd), lambda i: (i, _z())),
            pl.BlockSpec((2, hd), lambda i: (_z(), _z())),
            pl.BlockSpec((hd, hd), lambda i: (_z(), _z())),
            pl.BlockSpec((1, hd), lambda i: (_z(), _z())),
        ],
        out_specs=[
            pl.BlockSpec((bm, hd), lambda i: (i, _z())),
            pl.BlockSpec((bm, hd), lambda i: (i, _z())),
        ],
        out_shape=[
            jax.ShapeDtypeStruct((m, hd), F32),
            jax.ShapeDtypeStruct((m, hd), F32),
        ],
    )(t, e_in, prm, w, b.reshape(1, hd))


def _epost_body(t_ref, ein_ref, prm_ref, eo_ref):
    scale = prm_ref[0:1, :]
    beta = prm_ref[1:2, :]
    eo_ref[...] = jnp.maximum(t_ref[...] * scale + beta, 0.0) + ein_ref[...]


def _epost(t, e_in, prm, bm):
    m, hd = t.shape
    return pl.pallas_call(
        _epost_body,
        grid=(m // bm,),
        in_specs=[
            pl.BlockSpec((bm, hd), lambda i: (i, _z())),
            pl.BlockSpec((bm, hd), lambda i: (i, _z())),
            pl.BlockSpec((2, hd), lambda i: (_z(), _z())),
        ],
        out_specs=pl.BlockSpec((bm, hd), lambda i: (i, _z())),
        out_shape=jax.ShapeDtypeStruct((m, hd), F32),
    )(t, e_in, prm)


def _mm4_body(h_ref, w_ref, b_ref, db_ref, ea_ref):
    y = jnp.dot(h_ref[...], w_ref[...], preferred_element_type=F32) + b_ref[...]
    hdim = y.shape[1] // 4
    db_ref[...] = y[:, : 2 * hdim]
    ea_ref[...] = y[:, 2 * hdim :]


def _mm4(h, w_cat, b_cat, bm):
    m, k = h.shape
    hd = w_cat.shape[1] // 4
    return pl.pallas_call(
        _mm4_body,
        grid=(m // bm,),
        in_specs=[
            pl.BlockSpec((bm, k), lambda i: (i, _z())),
            pl.BlockSpec((k, 4 * hd), lambda i: (_z(), _z())),
            pl.BlockSpec((1, 4 * hd), lambda i: (_z(), _z())),
        ],
        out_specs=[
            pl.BlockSpec((bm, 2 * hd), lambda i: (i, _z())),
            pl.BlockSpec((bm, 2 * hd), lambda i: (i, _z())),
        ],
        out_shape=[
            jax.ShapeDtypeStruct((m, 2 * hd), F32),
            jax.ShapeDtypeStruct((m, 2 * hd), F32),
        ],
    )(h, w_cat, b_cat.reshape(1, 4 * hd))


def _update_body(ea_ref, acc_ref, hin_ref, sb_ref, ho_ref):
    hd = hin_ref.shape[1]
    ah = ea_ref[:, hd:]
    acc0 = acc_ref[0]
    acc1 = acc_ref[1]
    num = acc0[:, :hd] + acc1[:, :hd]
    den = acc0[:, hd:] + acc1[:, hd:]
    hn = ah + num / (den + 1e-6)
    scale = sb_ref[0:1, :]
    beta = sb_ref[1:2, :]
    ho_ref[...] = jnp.maximum(hn * scale + beta, 0.0) + hin_ref[...]


def _update(ea, accp, h_in, sb, bm):
    m, hd = h_in.shape
    return pl.pallas_call(
        _update_body,
        grid=(m // bm,),
        in_specs=[
            pl.BlockSpec((bm, 2 * hd), lambda i: (i, _z())),
            pl.BlockSpec((2, bm, 2 * hd), lambda i: (_z(), i, _z())),
            pl.BlockSpec((bm, hd), lambda i: (i, _z())),
            pl.BlockSpec((2, hd), lambda i: (_z(), _z())),
        ],
        out_specs=pl.BlockSpec((bm, hd), lambda i: (i, _z())),
        out_shape=jax.ShapeDtypeStruct((m, hd), F32),
    )(ea, accp, h_in, sb)


@functools.lru_cache(maxsize=None)
def _make_edge_kernel(n_edges, n_nodes, hd):
    epw = n_edges // NUM_WORKERS
    chunk = 40
    n_chunks = epw // chunk
    assert epw % chunk == 0 and chunk % 8 == 0 and n_chunks % 2 == 0
    rbase = (n_nodes // (16 * NUM_SUBCORES)) * 16
    zchunks_base = rbase // 16
    zchunks_last = (n_nodes - rbase * (NUM_SUBCORES - 1)) // 16
    assert n_nodes % 16 == 0

    mesh = plsc.VectorSubcoreMesh(core_axis_name="c", subcore_axis_name="s")

    def body(src_h, dst_h, ce_h, db_h, ea_h,
             eout_h, accp_h,
             srcv0, srcv1, dstv0, dstv1, dsts0, dsts1, dbv0, dbv1, eav0, eav1,
             cev0, cev1, contrib0, contrib1, accs,
             sin0, sin1, ssc0, ssc1):
        i32 = jnp.int32
        c = lax.axis_index("c").astype(i32)
        s = lax.axis_index("s").astype(i32)
        w = c * i32(NUM_SUBCORES) + s

        def zrow(r, carry):
            z = jnp.zeros((16,), F32)
            for j in range(2 * hd // 16):
                contrib0[r, pl.ds(j * 16, 16)] = z
            return carry

        lax.fori_loop(i32(0), i32(16), zrow, None)
        row0 = s * i32(rbase)
        nz = jnp.where(s == i32(NUM_SUBCORES - 1),
                       i32(zchunks_last), i32(zchunks_base))

        def zcopy(k, carry):
            pltpu.sync_copy(
                contrib0.at[pl.ds(0, 16)],
                accs.at[pl.ds(row0 + k * i32(16), 16)],
            )
            return carry

        lax.fori_loop(i32(0), nz, zcopy, None)
        plsc.subcore_barrier()

        base_w = w * i32(epw)
        bufs = ((srcv0, dstv0, dsts0, dbv0, eav0, cev0, contrib0, sin0, ssc0),
                (srcv1, dstv1, dsts1, dbv1, eav1, cev1, contrib1, sin1, ssc1))

        def issue_inputs(t, bset):
            srcv, dstv, _, dbv, eav, cev, _, sin, _ = bset
            base = pl.multiple_of(base_w + t * i32(chunk), 8)
            pltpu.sync_copy(src_h.at[pl.ds(base, chunk)], srcv)
            pltpu.sync_copy(dst_h.at[pl.ds(base, chunk)], dstv)
            pltpu.async_copy(ce_h.at[pl.ds(base, chunk)], cev, sin)
            pltpu.async_copy(db_h.at[srcv], dbv, sin)
            pltpu.async_copy(ea_h.at[dstv], eav, sin)

        def wait_inputs(t, bset):
            srcv, dstv, _, dbv, eav, cev, _, sin, _ = bset
            base = pl.multiple_of(base_w + t * i32(chunk), 8)
            pltpu.make_async_copy(ce_h.at[pl.ds(base, chunk)], cev, sin).wait()
            pltpu.make_async_copy(db_h.at[srcv], dbv, sin).wait()
            pltpu.make_async_copy(ea_h.at[dstv], eav, sin).wait()

        def compute_store(t, bset, first):
            _, dstv, dsts, dbv, eav, cev, contrib, _, ssc = bset
            base = pl.multiple_of(base_w + t * i32(chunk), 8)

            @pl.when(jnp.logical_not(first))
            def _():
                pltpu.make_async_copy(contrib, accs.at[dsts], ssc).wait()

            def row(r, rc):
                for j in range(hd // 16):
                    o = j * 16
                    dh = dbv[r, pl.ds(o, 16)]
                    bh = dbv[r, pl.ds(hd + o, 16)]
                    ehg = eav[r, pl.ds(o, 16)]
                    ce = cev[r, pl.ds(o, 16)]
                    tv = ce + dh + ehg
                    sg = 1.0 / (1.0 + jnp.exp(-tv))
                    contrib[r, pl.ds(o, 16)] = sg * bh
                    contrib[r, pl.ds(hd + o, 16)] = sg
                    cev[r, pl.ds(o, 16)] = tv
                return rc

            lax.fori_loop(i32(0), i32(chunk), row, None)

            for off in range(0, chunk - 15, 16):
                dsts[pl.ds(off, 16)] = dstv[pl.ds(off, 16)]
            if chunk % 16:
                off = chunk - 16
                dsts[pl.ds(off, 16)] = dstv[pl.ds(off, 16)]

            pltpu.sync_copy(cev, eout_h.at[pl.ds(base, chunk)])
            pltpu.async_copy(contrib, accs.at[dsts], ssc, add=True)

        issue_inputs(i32(0), bufs[0])

        def pair(p, carry):
            t0 = p * i32(2)
            t1 = t0 + i32(1)
            t2 = jnp.minimum(t0 + i32(2), i32(n_chunks - 1))
            issue_inputs(t1, bufs[1])
            wait_inputs(t0, bufs[0])
            compute_store(t0, bufs[0], p == i32(0))
            issue_inputs(t2, bufs[0])
            wait_inputs(t1, bufs[1])
            compute_store(t1, bufs[1], p == i32(0))
            return carry

        lax.fori_loop(i32(0), i32(n_chunks // 2), pair, None)

        wait_inputs(i32(n_chunks - 1), bufs[0])
        pltpu.make_async_copy(contrib0, accs.at[dsts0], ssc0).wait()
        pltpu.make_async_copy(contrib1, accs.at[dsts1], ssc1).wait()
        plsc.subcore_barrier()

        def wcopy(k, carry):
            r = row0 + k * i32(16)
            pltpu.sync_copy(
                accs.at[pl.ds(r, 16)],
                accp_h.at[c, pl.ds(r, 16)],
            )
            return carry

        lax.fori_loop(i32(0), nz, wcopy, None)

    return pl.kernel(
        body,
        mesh=mesh,
        out_type=[
            jax.ShapeDtypeStruct((n_edges, hd), F32),
            jax.ShapeDtypeStruct((NUM_CORES, n_nodes, 2 * hd), F32),
        ],
        scratch_types=[
            pltpu.VMEM((chunk,), jnp.int32),
            pltpu.VMEM((chunk,), jnp.int32),
            pltpu.VMEM((chunk,), jnp.int32),
            pltpu.VMEM((chunk,), jnp.int32),
            pltpu.VMEM((chunk,), jnp.int32),
            pltpu.VMEM((chunk,), jnp.int32),
            pltpu.VMEM((chunk, 2 * hd), F32),
            pltpu.VMEM((chunk, 2 * hd), F32),
            pltpu.VMEM((chunk, 2 * hd), F32),
            pltpu.VMEM((chunk, 2 * hd), F32),
            pltpu.VMEM((chunk, hd), F32),
            pltpu.VMEM((chunk, hd), F32),
            pltpu.VMEM((chunk, 2 * hd), F32),
            pltpu.VMEM((chunk, 2 * hd), F32),
            pltpu.VMEM_SHARED((n_nodes, 2 * hd), F32),
            pltpu.SemaphoreType.DMA,
            pltpu.SemaphoreType.DMA,
            pltpu.SemaphoreType.DMA,
            pltpu.SemaphoreType.DMA,
        ],
    )


def kernel(h, e, edge_index, emb_h_W, emb_h_b, emb_e_W, emb_e_b,
           layer_W, layer_b, bn_gamma, bn_beta):
    n_nodes = h.shape[0]
    n_edges = e.shape[0]
    hd = emb_h_W.shape[1]
    n_layers = layer_W.shape[0]

    src = edge_index[0].astype(jnp.int32)
    dst = edge_index[1].astype(jnp.int32)

    out_dtype = jnp.result_type(h.dtype, emb_h_W.dtype)
    h = h.astype(F32)
    e = e.astype(F32)
    emb_h_W = emb_h_W.astype(F32)
    emb_e_W = emb_e_W.astype(F32)
    emb_h_b = emb_h_b.astype(F32)
    emb_e_b = emb_e_b.astype(F32)
    layer_W = layer_W.astype(F32)
    layer_b = layer_b.astype(F32)
    bn_gamma = bn_gamma.astype(F32)
    bn_beta = bn_beta.astype(F32)

    h = _affine(h, emb_h_W, emb_h_b, 2000)
    e = _affine(e, emb_e_W, emb_e_b, 4000)

    bn_inv = 1.0 / jnp.sqrt(jnp.float32(1.0 + 1e-5))
    edge_fn = _make_edge_kernel(n_edges, n_nodes, hd)

    t_prev = None
    ein_prev = None
    prm_prev = None
    for l in range(n_layers):
        wl = layer_W[l]
        bl = layer_b[l]
        w_cat = jnp.concatenate([wl[2], wl[1], wl[3], wl[0]], axis=1)
        b_cat = jnp.concatenate([bl[2], bl[1], bl[3], bl[0]])
        db, ea = _mm4(h, w_cat, b_cat, 2000)

        if l == 0:
            ce = _affine(e, wl[4], bl[4], 4000)
            ein = e
        else:
            ce, e_mat = _ce_fused(t_prev, ein_prev, prm_prev, wl[4], bl[4], 4000)
            ein = e_mat

        t_new, accp = edge_fn(src, dst, ce, db, ea)

        sb_h = jnp.stack([bn_gamma[l, 0] * bn_inv, bn_beta[l, 0]])
        h = _update(ea, accp, h, sb_h, 2000)

        t_prev = t_new
        ein_prev = ein
        prm_prev = jnp.stack([bn_gamma[l, 1] * bn_inv, bn_beta[l, 1]])

    e_final = _epost(t_prev, ein_prev, prm_prev, 4000)
    return h.astype(out_dtype), e_final.astype(out_dtype)


# speedup vs baseline: 92.7748x; 1.1081x over previous
import functools

import jax
import jax.numpy as jnp
from jax import lax
from jax.experimental import pallas as pl
from jax.experimental.pallas import tpu as pltpu
from jax.experimental.pallas import tpu_sc as plsc

F32 = jnp.float32


def _z():
    return jnp.int32(0)

NUM_CORES = 2
NUM_SUBCORES = 16
NUM_WORKERS = NUM_CORES * NUM_SUBCORES


def _affine_body(x_ref, w_ref, b_ref, o_ref):
    o_ref[...] = (
        jnp.dot(x_ref[...], w_ref[...], preferred_element_type=F32) + b_ref[...]
    )


def _affine(x, w, b, bm):
    m, k = x.shape
    n = w.shape[1]
    return pl.pallas_call(
        _affine_body,
        grid=(m // bm,),
        in_specs=[
            pl.BlockSpec((bm, k), lambda i: (i, _z())),
            pl.BlockSpec((k, n), lambda i: (_z(), _z())),
            pl.BlockSpec((1, n), lambda i: (_z(), _z())),
        ],
        out_specs=pl.BlockSpec((bm, n), lambda i: (i, _z())),
        out_shape=jax.ShapeDtypeStruct((m, n), F32),
    )(x, w, b.reshape(1, n))


def _emb2_body(x_ref, w1_ref, b1_ref, w2_ref, b2_ref, emb_ref, ce_ref):
    x = x_ref[...]
    emb_ref[...] = (
        jnp.dot(x, w1_ref[...], preferred_element_type=F32) + b1_ref[...]
    )
    ce_ref[...] = (
        jnp.dot(x, w2_ref[...], preferred_element_type=F32) + b2_ref[...]
    )


def _emb2(x, w1, b1, w2, b2, bm):
    m, k = x.shape
    n = w1.shape[1]
    return pl.pallas_call(
        _emb2_body,
        grid=(m // bm,),
        in_specs=[
            pl.BlockSpec((bm, k), lambda i: (i, _z())),
            pl.BlockSpec((k, n), lambda i: (_z(), _z())),
            pl.BlockSpec((1, n), lambda i: (_z(), _z())),
            pl.BlockSpec((k, n), lambda i: (_z(), _z())),
            pl.BlockSpec((1, n), lambda i: (_z(), _z())),
        ],
        out_specs=[
            pl.BlockSpec((bm, n), lambda i: (i, _z())),
            pl.BlockSpec((bm, n), lambda i: (i, _z())),
        ],
        out_shape=[
            jax.ShapeDtypeStruct((m, n), F32),
            jax.ShapeDtypeStruct((m, n), F32),
        ],
    )(x, w1, b1.reshape(1, n), w2, b2.reshape(1, n))


def _cef_body(t_ref, ein_ref, prm_ref, w_ref, b_ref, ce_ref, eo_ref):
    scale = prm_ref[0:1, :]
    beta = prm_ref[1:2, :]
    eo = jnp.maximum(t_ref[...] * scale + beta, 0.0) + ein_ref[...]
    eo_ref[...] = eo
    ce_ref[...] = jnp.dot(eo, w_ref[...], preferred_element_type=F32) + b_ref[...]


def _ce_fused(t, e_in, prm, w, b, bm):
    m, hd = t.shape
    return pl.pallas_call(
        _cef_body,
        grid=(m // bm,),
        in_specs=[
            pl.BlockSpec((bm, hd), lambda i: (i, _z())),
            pl.BlockSpec((bm, hd), lambda i: (i, _z())),
            pl.BlockSpec((2, hd), lambda i: (_z(), _z())),
            pl.BlockSpec((hd, hd), lambda i: (_z(), _z())),
            pl.BlockSpec((1, hd), lambda i: (_z(), _z())),
        ],
        out_specs=[
            pl.BlockSpec((bm, hd), lambda i: (i, _z())),
            pl.BlockSpec((bm, hd), lambda i: (i, _z())),
        ],
        out_shape=[
            jax.ShapeDtypeStruct((m, hd), F32),
            jax.ShapeDtypeStruct((m, hd), F32),
        ],
    )(t, e_in, prm, w, b.reshape(1, hd))


def _epost_body(t_ref, ein_ref, prm_ref, eo_ref):
    scale = prm_ref[0:1, :]
    beta = prm_ref[1:2, :]
    eo_ref[...] = jnp.maximum(t_ref[...] * scale + beta, 0.0) + ein_ref[...]


def _epost(t, e_in, prm, bm):
    m, hd = t.shape
    return pl.pallas_call(
        _epost_body,
        grid=(m // bm,),
        in_specs=[
            pl.BlockSpec((bm, hd), lambda i: (i, _z())),
            pl.BlockSpec((bm, hd), lambda i: (i, _z())),
            pl.BlockSpec((2, hd), lambda i: (_z(), _z())),
        ],
        out_specs=pl.BlockSpec((bm, hd), lambda i: (i, _z())),
        out_shape=jax.ShapeDtypeStruct((m, hd), F32),
    )(t, e_in, prm)


def _mm4_body(h_ref, w_ref, b_ref, db_ref, ea_ref):
    y = jnp.dot(h_ref[...], w_ref[...], preferred_element_type=F32) + b_ref[...]
    hdim = y.shape[1] // 4
    db_ref[...] = y[:, : 2 * hdim]
    ea_ref[...] = y[:, 2 * hdim :]


def _mm4(h, w_cat, b_cat, bm):
    m, k = h.shape
    hd = w_cat.shape[1] // 4
    return pl.pallas_call(
        _mm4_body,
        grid=(m // bm,),
        in_specs=[
            pl.BlockSpec((bm, k), lambda i: (i, _z())),
            pl.BlockSpec((k, 4 * hd), lambda i: (_z(), _z())),
            pl.BlockSpec((1, 4 * hd), lambda i: (_z(), _z())),
        ],
        out_specs=[
            pl.BlockSpec((bm, 2 * hd), lambda i: (i, _z())),
            pl.BlockSpec((bm, 2 * hd), lambda i: (i, _z())),
        ],
        out_shape=[
            jax.ShapeDtypeStruct((m, 2 * hd), F32),
            jax.ShapeDtypeStruct((m, 2 * hd), F32),
        ],
    )(h, w_cat, b_cat.reshape(1, 4 * hd))


def _update_body(ea_ref, acc_ref, hin_ref, sb_ref, ho_ref):
    hd = hin_ref.shape[1]
    ah = ea_ref[:, hd:]
    acc0 = acc_ref[0]
    acc1 = acc_ref[1]
    num = acc0[:, :hd] + acc1[:, :hd]
    den = acc0[:, hd:] + acc1[:, hd:]
    hn = ah + num / (den + 1e-6)
    scale = sb_ref[0:1, :]
    beta = sb_ref[1:2, :]
    ho_ref[...] = jnp.maximum(hn * scale + beta, 0.0) + hin_ref[...]


def _update(ea, accp, h_in, sb, bm):
    m, hd = h_in.shape
    return pl.pallas_call(
        _update_body,
        grid=(m // bm,),
        in_specs=[
            pl.BlockSpec((bm, 2 * hd), lambda i: (i, _z())),
            pl.BlockSpec((2, bm, 2 * hd), lambda i: (_z(), i, _z())),
            pl.BlockSpec((bm, hd), lambda i: (i, _z())),
            pl.BlockSpec((2, hd), lambda i: (_z(), _z())),
        ],
        out_specs=pl.BlockSpec((bm, hd), lambda i: (i, _z())),
        out_shape=jax.ShapeDtypeStruct((m, hd), F32),
    )(ea, accp, h_in, sb)


@functools.lru_cache(maxsize=None)
def _make_edge_kernel(n_edges, n_nodes, hd):
    epw = n_edges // NUM_WORKERS
    chunk = 40
    n_chunks = epw // chunk
    sb_chunks = 50
    n_sb = n_chunks // sb_chunks
    assert epw % chunk == 0 and chunk % 8 == 0 and n_chunks % 2 == 0
    assert n_chunks % sb_chunks == 0 and sb_chunks % 2 == 0
    rbase = (n_nodes // (16 * NUM_SUBCORES)) * 16
    zchunks_base = rbase // 16
    zchunks_last = (n_nodes - rbase * (NUM_SUBCORES - 1)) // 16
    assert n_nodes % 16 == 0

    mesh = plsc.VectorSubcoreMesh(core_axis_name="c", subcore_axis_name="s")

    def body(src_h, dst_h, ce_h, db_h, ea_h,
             eout_h, accp_h,
             src_big, dst_big, dsts0, dsts1, dbv0, dbv1, eav0, eav1,
             cev0, cev1, contrib0, contrib1, accs,
             sin0, sin1, ssc0, ssc1, seo0, seo1):
        i32 = jnp.int32
        c = lax.axis_index("c").astype(i32)
        s = lax.axis_index("s").astype(i32)
        w = c * i32(NUM_SUBCORES) + s

        def zrow(r, carry):
            z = jnp.zeros((16,), F32)
            for j in range(2 * hd // 16):
                contrib0[r, pl.ds(j * 16, 16)] = z
            return carry

        lax.fori_loop(i32(0), i32(16), zrow, None)
        row0 = s * i32(rbase)
        nz = jnp.where(s == i32(NUM_SUBCORES - 1),
                       i32(zchunks_last), i32(zchunks_base))

        def zcopy(k, carry):
            pltpu.sync_copy(
                contrib0.at[pl.ds(0, 16)],
                accs.at[pl.ds(row0 + k * i32(16), 16)],
            )
            return carry

        lax.fori_loop(i32(0), nz, zcopy, None)
        plsc.subcore_barrier()

        base_w = w * i32(epw)
        bufs = ((dsts0, dbv0, eav0, cev0, contrib0, sin0, ssc0, seo0),
                (dsts1, dbv1, eav1, cev1, contrib1, sin1, ssc1, seo1))

        def issue_inputs(t, k, bset, wait_eo):
            _, dbv, eav, cev, _, sin, _, seo = bset
            base = pl.multiple_of(base_w + t * i32(chunk), 8)
            off = k * i32(chunk)

            @pl.when(wait_eo)
            def _():
                pltpu.make_async_copy(
                    cev, eout_h.at[pl.ds(base, chunk)], seo).wait()

            pltpu.async_copy(ce_h.at[pl.ds(base, chunk)], cev, sin)
            pltpu.async_copy(db_h.at[src_big.at[pl.ds(off, chunk)]], dbv, sin)
            pltpu.async_copy(ea_h.at[dst_big.at[pl.ds(off, chunk)]], eav, sin)

        def wait_inputs(t, k, bset):
            _, dbv, eav, cev, _, sin, _, _ = bset
            base = pl.multiple_of(base_w + t * i32(chunk), 8)
            off = k * i32(chunk)
            pltpu.make_async_copy(ce_h.at[pl.ds(base, chunk)], cev, sin).wait()
            pltpu.make_async_copy(
                db_h.at[src_big.at[pl.ds(off, chunk)]], dbv, sin).wait()
            pltpu.make_async_copy(
                ea_h.at[dst_big.at[pl.ds(off, chunk)]], eav, sin).wait()

        def compute_store(t, k, bset, first):
            dsts, dbv, eav, cev, contrib, _, ssc, seo = bset
            base = pl.multiple_of(base_w + t * i32(chunk), 8)
            off = k * i32(chunk)

            @pl.when(jnp.logical_not(first))
            def _():
                pltpu.make_async_copy(contrib, accs.at[dsts], ssc).wait()

            def row(r, rc):
                for j in range(hd // 16):
                    o = j * 16
                    dh = dbv[r, pl.ds(o, 16)]
                    bh = dbv[r, pl.ds(hd + o, 16)]
                    ehg = eav[r, pl.ds(o, 16)]
                    ce = cev[r, pl.ds(o, 16)]
                    tv = ce + dh + ehg
                    sg = 1.0 / (1.0 + jnp.exp(-tv))
                    contrib[r, pl.ds(o, 16)] = sg * bh
                    contrib[r, pl.ds(hd + o, 16)] = sg
                    cev[r, pl.ds(o, 16)] = tv
                return rc

            lax.fori_loop(i32(0), i32(chunk), row, None)

            pltpu.async_copy(cev, eout_h.at[pl.ds(base, chunk)], seo)

            for o2 in range(0, chunk - 15, 16):
                dsts[pl.ds(o2, 16)] = dst_big[pl.ds(off + o2, 16)]
            if chunk % 16:
                o2 = chunk - 16
                dsts[pl.ds(o2, 16)] = dst_big[pl.ds(off + o2, 16)]

            pltpu.async_copy(contrib, accs.at[dsts], ssc, add=True)

        def superblock(sb, carry):
            sb_base = pl.multiple_of(base_w + sb * i32(sb_chunks * chunk), 8)
            pltpu.sync_copy(src_h.at[pl.ds(sb_base, sb_chunks * chunk)], src_big)
            pltpu.sync_copy(dst_h.at[pl.ds(sb_base, sb_chunks * chunk)], dst_big)
            t_base = sb * i32(sb_chunks)
            not_first_sb = sb > i32(0)

            issue_inputs(t_base, i32(0), bufs[0], not_first_sb)

            def pair(p, carry2):
                k0 = p * i32(2)
                k1 = k0 + i32(1)
                first = jnp.logical_and(p == i32(0),
                                        jnp.logical_not(not_first_sb))
                issue_inputs(t_base + k1, k1, bufs[1],
                             jnp.logical_or(p > i32(0), not_first_sb))
                wait_inputs(t_base + k0, k0, bufs[0])
                compute_store(t_base + k0, k0, bufs[0], first)

                @pl.when(p < i32(sb_chunks // 2 - 1))
                def _():
                    issue_inputs(t_base + k0 + i32(2), k0 + i32(2), bufs[0],
                                 jnp.bool_(True))

                wait_inputs(t_base + k1, k1, bufs[1])
                compute_store(t_base + k1, k1, bufs[1], first)
                return carry2

            lax.fori_loop(i32(0), i32(sb_chunks // 2), pair, None)
            return carry

        lax.fori_loop(i32(0), i32(n_sb), superblock, None)

        pen = i32(n_chunks - 2)
        pen_base = pl.multiple_of(base_w + pen * i32(chunk), 8)
        pltpu.make_async_copy(
            cev0, eout_h.at[pl.ds(pen_base, chunk)], seo0).wait()
        last = i32(n_chunks - 1)
        last_base = pl.multiple_of(base_w + last * i32(chunk), 8)
        pltpu.make_async_copy(
            cev1, eout_h.at[pl.ds(last_base, chunk)], seo1).wait()
        pltpu.make_async_copy(contrib0, accs.at[dsts0], ssc0).wait()
        pltpu.make_async_copy(contrib1, accs.at[dsts1], ssc1).wait()
        plsc.subcore_barrier()

        def wcopy(k, carry):
            r = row0 + k * i32(16)
            pltpu.sync_copy(
                accs.at[pl.ds(r, 16)],
                accp_h.at[c, pl.ds(r, 16)],
            )
            return carry

        lax.fori_loop(i32(0), nz, wcopy, None)

    return pl.kernel(
        body,
        mesh=mesh,
        out_type=[
            jax.ShapeDtypeStruct((n_edges, hd), F32),
            jax.ShapeDtypeStruct((NUM_CORES, n_nodes, 2 * hd), F32),
        ],
        scratch_types=[
            pltpu.VMEM((sb_chunks * chunk,), jnp.int32),
            pltpu.VMEM((sb_chunks * chunk,), jnp.int32),
            pltpu.VMEM((chunk,), jnp.int32),
            pltpu.VMEM((chunk,), jnp.int32),
            pltpu.VMEM((chunk, 2 * hd), F32),
            pltpu.VMEM((chunk, 2 * hd), F32),
            pltpu.VMEM((chunk, 2 * hd), F32),
            pltpu.VMEM((chunk, 2 * hd), F32),
            pltpu.VMEM((chunk, hd), F32),
            pltpu.VMEM((chunk, hd), F32),
            pltpu.VMEM((chunk, 2 * hd), F32),
            pltpu.VMEM((chunk, 2 * hd), F32),
            pltpu.VMEM_SHARED((n_nodes, 2 * hd), F32),
            pltpu.SemaphoreType.DMA,
            pltpu.SemaphoreType.DMA,
            pltpu.SemaphoreType.DMA,
            pltpu.SemaphoreType.DMA,
            pltpu.SemaphoreType.DMA,
            pltpu.SemaphoreType.DMA,
        ],
    )


def kernel(h, e, edge_index, emb_h_W, emb_h_b, emb_e_W, emb_e_b,
           layer_W, layer_b, bn_gamma, bn_beta):
    n_nodes = h.shape[0]
    n_edges = e.shape[0]
    hd = emb_h_W.shape[1]
    n_layers = layer_W.shape[0]

    src = edge_index[0].astype(jnp.int32)
    dst = edge_index[1].astype(jnp.int32)

    out_dtype = jnp.result_type(h.dtype, emb_h_W.dtype)
    h = h.astype(F32)
    e = e.astype(F32)
    emb_h_W = emb_h_W.astype(F32)
    emb_e_W = emb_e_W.astype(F32)
    emb_h_b = emb_h_b.astype(F32)
    emb_e_b = emb_e_b.astype(F32)
    layer_W = layer_W.astype(F32)
    layer_b = layer_b.astype(F32)
    bn_gamma = bn_gamma.astype(F32)
    bn_beta = bn_beta.astype(F32)

    h = _affine(h, emb_h_W, emb_h_b, 2000)
    w_ce0 = emb_e_W @ layer_W[0, 4]
    b_ce0 = emb_e_b @ layer_W[0, 4] + layer_b[0, 4]
    e, ce0 = _emb2(e, emb_e_W, emb_e_b, w_ce0, b_ce0, 8000)

    bn_inv = 1.0 / jnp.sqrt(jnp.float32(1.0 + 1e-5))
    edge_fn = _make_edge_kernel(n_edges, n_nodes, hd)

    t_prev = None
    ein_prev = None
    prm_prev = None
    for l in range(n_layers):
        wl = layer_W[l]
        bl = layer_b[l]
        w_cat = jnp.concatenate([wl[2], wl[1], wl[3], wl[0]], axis=1)
        b_cat = jnp.concatenate([bl[2], bl[1], bl[3], bl[0]])
        db, ea = _mm4(h, w_cat, b_cat, 2000)

        if l == 0:
            ce = ce0
            ein = e
        else:
            ce, e_mat = _ce_fused(t_prev, ein_prev, prm_prev, wl[4], bl[4], 8000)
            ein = e_mat

        t_new, accp = edge_fn(src, dst, ce, db, ea)

        sb_h = jnp.stack([bn_gamma[l, 0] * bn_inv, bn_beta[l, 0]])
        h = _update(ea, accp, h, sb_h, 2000)

        t_prev = t_new
        ein_prev = ein
        prm_prev = jnp.stack([bn_gamma[l, 1] * bn_inv, bn_beta[l, 1]])

    e_final = _epost(t_prev, ein_prev, prm_prev, 8000)
    return h.astype(out_dtype), e_final.astype(out_dtype)
